# baseline (speedup 1.0000x reference)
import functools

import jax
import jax.numpy as jnp
from jax import lax
from jax.experimental import pallas as pl
from jax.experimental.pallas import tpu as pltpu
from jax.experimental.pallas import tpu_sc as plsc

B = 16384
F = 26
E = 16
V = 1000000
NC, NS = 2, 16
NW = NC * NS
SPW = B // NW
RPW = SPW * F
G = 128
GPW = RPW // G

TW_BLK = 262144


def _tw_body(t_ref, w_ref, o_ref):
    o_ref[...] = jnp.sum(t_ref[...] * w_ref[...], axis=0)


def _fold_weight(table_t, weight):
    grid = (V + TW_BLK - 1) // TW_BLK
    return pl.pallas_call(
        _tw_body,
        grid=(grid,),
        in_specs=[
            pl.BlockSpec((E, TW_BLK), lambda i: (0, i)),
            pl.BlockSpec((E, 1), lambda i: (0, 0)),
        ],
        out_specs=pl.BlockSpec((TW_BLK,), lambda i: (i,)),
        out_shape=jax.ShapeDtypeStruct((V,), jnp.float32),
    )(table_t, weight)


def _sc_body(idx_hbm, val_hbm, tw_hbm, b_hbm, out_hbm,
             idx_a, idx_b, val_a, g_v, pre_v, b_v, sem):
    wid = lax.axis_index("s") * NC + lax.axis_index("c")

    pltpu.sync_copy(idx_hbm.at[pl.ds(wid * SPW, SPW)], idx_a)
    pltpu.sync_copy(val_hbm.at[pl.ds(wid * SPW, SPW)], val_a)
    pltpu.sync_copy(b_hbm, b_v)

    lanes = lax.iota(jnp.int32, E)
    bvec = b_v[...]
    inv_f = jnp.float32(1.0 / F)
    onehot = [lanes == k for k in range(E)]
    tail = lanes >= (2 * E - F)

    def repack_body(s, carry):
        ia = idx_a[s, pl.ds(0, E)]
        ib = idx_a[s, pl.ds(F - E, E)]
        p0 = s * F
        idx_b[pl.ds(p0, E)] = ia
        idx_b[pl.ds(p0 + (F - E), E)] = ib
        return carry

    lax.fori_loop(0, SPW, repack_body, 0)

    copies = []
    for j in range(GPW):
        cp = pltpu.make_async_copy(
            tw_hbm.at[idx_b.at[pl.ds(j * G, G)]],
            g_v.at[pl.ds(j * G, G)],
            sem,
        )
        cp.start()
        copies.append(cp)
    for cp in copies:
        cp.wait()

    def group_body(g, carry):
        svec = jnp.zeros((E,), jnp.float32)
        for k in range(E):
            s = g * E + k
            m0 = s * F
            va = val_a[s, pl.ds(0, E)]
            vb = val_a[s, pl.ds(F - E, E)]
            ga = g_v[pl.ds(m0, E)]
            gb = g_v[pl.ds(m0 + (F - E), E)]
            t = va * ga + jnp.where(tail, vb * gb, 0.0)
            for d in (8, 4, 2, 1):
                perm = lanes ^ d
                t = t + t.at[perm].get(mode="promise_in_bounds")
            svec = jnp.where(onehot[k], t, svec)
        pre_v[pl.ds(g * E, E)] = svec
        return carry

    lax.fori_loop(0, SPW // E, group_body, 0)

    for i in range(SPW // E):
        x = pre_v[pl.ds(i * E, E)] * inv_f + bvec
        pre_v[pl.ds(i * E, E)] = 1.0 / (1.0 + jnp.exp(-x))

    pltpu.sync_copy(pre_v, out_hbm.at[pl.ds(wid * SPW, SPW)])


@jax.jit
def _lr(feat_index, feat_value, emb_table, weight, bias):
    tw = _fold_weight(emb_table.T, weight)
    b16 = jnp.broadcast_to(bias, (E,))
    run = pl.kernel(
        _sc_body,
        out_type=jax.ShapeDtypeStruct((B,), jnp.float32),
        mesh=plsc.VectorSubcoreMesh(core_axis_name="c", subcore_axis_name="s"),
        scratch_types=[
            pltpu.VMEM((SPW, F), jnp.int32),
            pltpu.VMEM((RPW,), jnp.int32),
            pltpu.VMEM((SPW, F), jnp.float32),
            pltpu.VMEM((RPW,), jnp.float32),
            pltpu.VMEM((SPW,), jnp.float32),
            pltpu.VMEM((E,), jnp.float32),
            pltpu.SemaphoreType.DMA,
        ],
        compiler_params=pltpu.CompilerParams(use_tc_tiling_on_sc=False),
    )
    out = run(feat_index, feat_value, tw, b16)
    return out.reshape(B, 1)


def kernel(feat_index, feat_value, emb_table, weight, bias):
    return _lr(feat_index, feat_value, emb_table, weight, bias)


# speedup vs baseline: 1.5149x; 1.5149x over previous
import jax
import jax.numpy as jnp
from jax import lax
from jax.experimental import pallas as pl
from jax.experimental.pallas import tpu as pltpu
from jax.experimental.pallas import tpu_sc as plsc

B = 16384
F = 26
E = 16
V = 1000000
NC, NS = 2, 16
NW = NC * NS
SPW = B // NW

TW_BLK = 131072


def _tw_body(t_ref, w_ref, o_ref):
    o_ref[...] = jnp.sum(t_ref[...] * w_ref[...], axis=0)


def _fold_weight(table_t, weight):
    grid = (V + TW_BLK - 1) // TW_BLK
    return pl.pallas_call(
        _tw_body,
        grid=(grid,),
        in_specs=[
            pl.BlockSpec((E, TW_BLK), lambda i: (0, i)),
            pl.BlockSpec((E, 1), lambda i: (0, 0)),
        ],
        out_specs=pl.BlockSpec((TW_BLK,), lambda i: (i,)),
        out_shape=jax.ShapeDtypeStruct((V,), jnp.float32),
    )(table_t, weight)


def _sc_body(idx_hbm, val_hbm, tw_hbm, b_hbm, out_hbm,
             idx_v, val_v, g_v, out_v, b_v, sem):
    wid = lax.axis_index("s") * NC + lax.axis_index("c")

    pltpu.sync_copy(idx_hbm.at[:, pl.ds(wid * SPW, SPW)], idx_v)
    pltpu.sync_copy(val_hbm.at[:, pl.ds(wid * SPW, SPW)], val_v)
    pltpu.sync_copy(b_hbm, b_v)

    bvec = b_v[...]
    inv_f = jnp.float32(1.0 / F)

    copies = []
    for f in range(F):
        cp = pltpu.make_async_copy(
            tw_hbm.at[idx_v.at[f]],
            g_v.at[f],
            sem,
        )
        cp.start()
        copies.append(cp)
    for cp in copies:
        cp.wait()

    def group_body(g, carry):
        col = pl.ds(g * E, E)
        acc = val_v[0, col] * g_v[0, col]
        for f in range(1, F):
            acc = acc + val_v[f, col] * g_v[f, col]
        x = acc * inv_f + bvec
        out_v[col] = 1.0 / (1.0 + jnp.exp(-x))
        return carry

    lax.fori_loop(0, SPW // E, group_body, 0)

    pltpu.sync_copy(out_v, out_hbm.at[pl.ds(wid * SPW, SPW)])


@jax.jit
def _lr(feat_index, feat_value, emb_table, weight, bias):
    tw = _fold_weight(emb_table.T, weight)
    b16 = jnp.broadcast_to(bias, (E,))
    run = pl.kernel(
        _sc_body,
        out_type=jax.ShapeDtypeStruct((B,), jnp.float32),
        mesh=plsc.VectorSubcoreMesh(core_axis_name="c", subcore_axis_name="s"),
        scratch_types=[
            pltpu.VMEM((F, SPW), jnp.int32),
            pltpu.VMEM((F, SPW), jnp.float32),
            pltpu.VMEM((F, SPW), jnp.float32),
            pltpu.VMEM((SPW,), jnp.float32),
            pltpu.VMEM((E,), jnp.float32),
            pltpu.SemaphoreType.DMA,
        ],
        compiler_params=pltpu.CompilerParams(use_tc_tiling_on_sc=False),
    )
    out = run(feat_index.T, feat_value.T, tw, b16)
    return out.reshape(B, 1)


def kernel(feat_index, feat_value, emb_table, weight, bias):
    return _lr(feat_index, feat_value, emb_table, weight, bias)


# speedup vs baseline: 1.7223x; 1.1369x over previous
import jax
import jax.numpy as jnp
from jax import lax
from jax.experimental import pallas as pl
from jax.experimental.pallas import tpu as pltpu
from jax.experimental.pallas import tpu_sc as plsc

B = 16384
F = 26
E = 16
V = 1000000
NC, NS = 2, 16
NW = NC * NS
SPW = B // NW
TWCH = 62504
VP = NS * TWCH

TW_BLK = 131072


def _tw_body(t_ref, w_ref, o_ref):
    o_ref[...] = jnp.sum(t_ref[...] * w_ref[...], axis=0)


def _fold_weight(table_t, weight):
    grid = (VP + TW_BLK - 1) // TW_BLK
    return pl.pallas_call(
        _tw_body,
        grid=(grid,),
        in_specs=[
            pl.BlockSpec((E, TW_BLK), lambda i: (0, i)),
            pl.BlockSpec((E, 1), lambda i: (0, 0)),
        ],
        out_specs=pl.BlockSpec((TW_BLK,), lambda i: (i,)),
        out_shape=jax.ShapeDtypeStruct((VP,), jnp.float32),
    )(table_t, weight)


def _sc_body(idx_hbm, val_hbm, tw_hbm, b_hbm, out_hbm,
             idx_v, val_v, g_v, out_v, b_v, tw_s, sem):
    sid = lax.axis_index("s")
    wid = sid * NC + lax.axis_index("c")

    base = sid * TWCH
    tw_cp = pltpu.make_async_copy(
        tw_hbm.at[pl.ds(base, TWCH)],
        tw_s.at[pl.ds(base, TWCH)],
        sem,
    )
    tw_cp.start()

    pltpu.sync_copy(idx_hbm.at[:, pl.ds(wid * SPW, SPW)], idx_v)
    pltpu.sync_copy(val_hbm.at[:, pl.ds(wid * SPW, SPW)], val_v)
    pltpu.sync_copy(b_hbm, b_v)

    bvec = b_v[...]
    inv_f = jnp.float32(1.0 / F)

    tw_cp.wait()
    plsc.subcore_barrier()

    copies = []
    for f in range(F):
        cp = pltpu.make_async_copy(
            tw_s.at[idx_v.at[f]],
            g_v.at[f],
            sem,
        )
        cp.start()
        copies.append(cp)
    for cp in copies:
        cp.wait()

    def group_body(g, carry):
        col = pl.ds(g * E, E)
        acc = val_v[0, col] * g_v[0, col]
        for f in range(1, F):
            acc = acc + val_v[f, col] * g_v[f, col]
        x = acc * inv_f + bvec
        out_v[col] = 1.0 / (1.0 + jnp.exp(-x))
        return carry

    lax.fori_loop(0, SPW // E, group_body, 0)

    pltpu.sync_copy(out_v, out_hbm.at[pl.ds(wid * SPW, SPW)])


@jax.jit
def _lr(feat_index, feat_value, emb_table, weight, bias):
    tw = _fold_weight(emb_table.T, weight)
    b16 = jnp.broadcast_to(bias, (E,))
    run = pl.kernel(
        _sc_body,
        out_type=jax.ShapeDtypeStruct((B,), jnp.float32),
        mesh=plsc.VectorSubcoreMesh(core_axis_name="c", subcore_axis_name="s"),
        scratch_types=[
            pltpu.VMEM((F, SPW), jnp.int32),
            pltpu.VMEM((F, SPW), jnp.float32),
            pltpu.VMEM((F, SPW), jnp.float32),
            pltpu.VMEM((SPW,), jnp.float32),
            pltpu.VMEM((E,), jnp.float32),
            pltpu.VMEM_SHARED((VP,), jnp.float32),
            pltpu.SemaphoreType.DMA,
        ],
        compiler_params=pltpu.CompilerParams(use_tc_tiling_on_sc=False),
    )
    out = run(feat_index.T, feat_value.T, tw, b16)
    return out.reshape(B, 1)


def kernel(feat_index, feat_value, emb_table, weight, bias):
    return _lr(feat_index, feat_value, emb_table, weight, bias)
